# baseline (speedup 1.0000x reference)
import dataclasses
import functools

import jax
import jax.numpy as jnp
from jax import lax
from jax.experimental import pallas as pl
from jax.experimental.pallas import tpu as pltpu
from jax.experimental.pallas import tpu_sc as plsc

_N = 10000
_E = 320000
_D = 128
_NC = 2
_NS = 16
_NW = _NC * _NS
_NPAD = 10240
_EPW = _E // _NW
_K = 128
_NCHUNK = -(-_EPW // _K)
_EPWP = _NCHUNK * _K
_RPT = _NPAD // _NS


def _sc_compiler_params():
    cp = pltpu.CompilerParams()
    if "needs_layout_passes" in pltpu.CompilerParams.__dataclass_fields__:
        cp = dataclasses.replace(cp, needs_layout_passes=False)
    return cp


def _sc_segment_sum(x, pk):
    mesh = plsc.VectorSubcoreMesh(core_axis_name="c", subcore_axis_name="s")

    @functools.partial(
        pl.kernel,
        compiler_params=_sc_compiler_params(),
        out_type=[
            jax.ShapeDtypeStruct((_NPAD, _D), jnp.float32),
            jax.ShapeDtypeStruct((_NPAD, _D), jnp.float32),
            jax.ShapeDtypeStruct((_NW * _NPAD,), jnp.float32),
        ],
        mesh=mesh,
        scratch_types=[
            pltpu.VMEM((2, _K), jnp.int32),
            pltpu.VMEM((2, _K), jnp.int32),
            pltpu.VMEM((2, _K), jnp.int32),
            pltpu.VMEM((2, _K), jnp.int32),
            pltpu.VMEM((_K, _D), jnp.float32),
            pltpu.VMEM((_K, _D), jnp.float32),
            pltpu.VMEM((_NPAD,), jnp.float32),
            pltpu.VMEM_SHARED((_NPAD, _D), jnp.float32),
            pltpu.SemaphoreType.DMA,
            pltpu.SemaphoreType.DMA,
            pltpu.SemaphoreType.DMA,
            pltpu.SemaphoreType.DMA,
            pltpu.SemaphoreType.DMA,
            pltpu.SemaphoreType.DMA,
            pltpu.SemaphoreType.DMA,
            pltpu.SemaphoreType.DMA,
            pltpu.SemaphoreType.DMA,
        ],
    )
    def sc_kernel(x_hbm, pk_hbm, sum0_out, sum1_out, cnt_out,
                  pidx0, pidx1, pidx2, pidx3, gbuf0, gbuf1, hist, acc,
                  si0, si1, si2, si3, sg0, sg1, ss0, ss1, sz):
        c = lax.axis_index("c")
        s = lax.axis_index("s")
        wid = c * _NS + s
        lo = s * _RPT
        cb = wid * _NCHUNK

        zero16 = jnp.zeros((16,), jnp.float32)
        one16 = jnp.ones((16,), jnp.float32)

        pidx = (pidx0, pidx1, pidx2, pidx3)
        si = (si0, si1, si2, si3)
        gset = ((gbuf0, sg0, ss0), (gbuf1, sg1, ss1))

        @pl.loop(0, _K)
        def _(r):
            for q in range(_D // 16):
                gbuf0.at[r, pl.ds(q * 16, 16)][...] = zero16

        for t in range(_RPT // _K):
            pltpu.async_copy(gbuf0, acc.at[pl.ds(lo + t * _K, _K)], sz)

        @pl.loop(0, _NPAD, step=16)
        def _(j):
            hist[pl.ds(j, 16)] = zero16

        for t in range(_RPT // _K):
            pltpu.make_async_copy(gbuf0, acc.at[pl.ds(lo + t * _K, _K)],
                                  sz).wait()

        plsc.subcore_barrier()

        def load_idx(m, sl):
            pltpu.async_copy(pk_hbm.at[cb + m], pidx[sl], si[sl])

        def wait_idx(m, sl):
            pltpu.make_async_copy(pk_hbm.at[cb + m], pidx[sl],
                                  si[sl]).wait()

        def gather(j, ph):
            gbuf, sg, _ = gset[ph % 2]
            pltpu.async_copy(x_hbm.at[pidx[ph].at[0]], gbuf, sg)

        def consume(ph):
            gbuf, sg, ss = gset[ph % 2]
            pltpu.make_async_copy(x_hbm.at[pidx[ph].at[0]], gbuf, sg).wait()
            pltpu.sync_copy(gbuf, acc.at[pidx[ph].at[1]], add=True)
            for q in range(_K // 16):
                idxv = pidx[ph][1, pl.ds(q * 16, 16)]
                plsc.addupdate_scatter(hist, [idxv], one16)

        def wait_scatter(ph):
            del ph

        pltpu.sync_copy(pk_hbm.at[cb + 0], pidx0)
        pltpu.sync_copy(pk_hbm.at[cb + 1], pidx1)
        gather(0, 0)
        load_idx(2, 2)
        gather(1, 1)
        load_idx(3, 3)
        consume(0)
        consume(1)
        wait_scatter(0)
        wait_idx(2, 2)
        gather(2, 2)

        def step(j, ph):
            consume(ph)
            wait_scatter((ph + 3) % 4)
            load_idx(j + 2, (ph + 2) % 4)
            wait_idx(j + 1, (ph + 1) % 4)
            gather(j + 1, (ph + 1) % 4)

        nsteady = (_NCHUNK - 7) // 4

        @pl.loop(0, nsteady)
        def _(t):
            j4 = 4 * t
            for k in range(4):
                step(j4 + 2 + k, (2 + k) % 4)

        step(_NCHUNK - 5, 2)
        step(_NCHUNK - 4, 3)
        step(_NCHUNK - 3, 0)
        consume(1)
        wait_scatter(0)
        wait_idx(_NCHUNK - 1, 2)
        gather(_NCHUNK - 1, 2)
        consume(2)
        wait_scatter(1)
        wait_scatter(2)

        plsc.subcore_barrier()

        pltpu.async_copy(hist, cnt_out.at[pl.ds(wid * _NPAD, _NPAD)], sz)

        def out_slice(j, gbuf):
            pltpu.sync_copy(acc.at[pl.ds(j, _K)], gbuf)

            @pl.when(c == 0)
            def _():
                pltpu.sync_copy(gbuf, sum0_out.at[pl.ds(j, _K)])

            @pl.when(c == 1)
            def _():
                pltpu.sync_copy(gbuf, sum1_out.at[pl.ds(j, _K)])

        @pl.loop(0, _RPT, step=_K)
        def _(j):
            out_slice(lo + j, gbuf0)

        pltpu.make_async_copy(hist, cnt_out.at[pl.ds(wid * _NPAD, _NPAD)],
                              sz).wait()

    return sc_kernel(x, pk)


def _tc_finish(x_pad, w_t, b2, sum0, sum1, cnt_t):
    blk = 1024

    def body(x_ref, wt_ref, b_ref, s0_ref, s1_ref, c_ref, o_ref):
        ssum = s0_ref[...] + s1_ref[...]
        cc = jnp.sum(c_ref[...], axis=1, keepdims=True)
        m = jnp.where(cc > 0.0, ssum / jnp.maximum(cc, 1.0), x_ref[...])
        o_ref[...] = jnp.dot(
            m, wt_ref[...], preferred_element_type=jnp.float32,
            precision=lax.Precision.HIGHEST,
        ) + b_ref[...]

    return pl.pallas_call(
        body,
        grid=(_NPAD // blk,),
        in_specs=[
            pl.BlockSpec((blk, _D), lambda i: (i, 0)),
            pl.BlockSpec((_D, _D), lambda i: (0, 0)),
            pl.BlockSpec((1, _D), lambda i: (0, 0)),
            pl.BlockSpec((blk, _D), lambda i: (i, 0)),
            pl.BlockSpec((blk, _D), lambda i: (i, 0)),
            pl.BlockSpec((blk, _NW), lambda i: (i, 0)),
        ],
        out_specs=pl.BlockSpec((blk, _D), lambda i: (i, 0)),
        out_shape=jax.ShapeDtypeStruct((_NPAD, _D), jnp.float32),
    )(x_pad, w_t, b2, sum0, sum1, cnt_t)


@jax.jit
def kernel(x, edge_index, W, b):
    row = edge_index[0]
    col = edge_index[1]
    pad = _EPWP - _EPW
    rowp = jnp.pad(row.reshape(_NW, _EPW), ((0, 0), (0, pad))
                   ).reshape(_NW, _NCHUNK, _K)
    colp = jnp.pad(col.reshape(_NW, _EPW), ((0, 0), (0, pad)),
                   constant_values=_NPAD - 1).reshape(_NW, _NCHUNK, _K)
    pk = jnp.stack([rowp, colp], axis=2).reshape(_NW * _NCHUNK, 2, _K)
    sum0, sum1, cnth = _sc_segment_sum(x, pk)
    cnt_t = cnth.reshape(_NW, _NPAD).T
    x_pad = jnp.pad(x, ((0, _NPAD - _N), (0, 0)))
    out_pad = _tc_finish(x_pad, W.T, b.reshape(1, _D), sum0, sum1, cnt_t)
    return out_pad[:_N]


# speedup vs baseline: 1.1559x; 1.1559x over previous
import dataclasses
import functools

import jax
import jax.numpy as jnp
from jax import lax
from jax.experimental import pallas as pl
from jax.experimental.pallas import tpu as pltpu
from jax.experimental.pallas import tpu_sc as plsc

_N = 10000
_E = 320000
_D = 128
_NC = 2
_NS = 16
_NW = _NC * _NS
_NPAD = 10240
_EPW = _E // _NW
_K = 128
_NCHUNK = -(-_EPW // _K)
_EPWP = _NCHUNK * _K
_RPT = _NPAD // _NS


def _sc_compiler_params():
    cp = pltpu.CompilerParams()
    if "needs_layout_passes" in pltpu.CompilerParams.__dataclass_fields__:
        cp = dataclasses.replace(cp, needs_layout_passes=False)
    return cp


def _sc_segment_sum(x, rowp, colp):
    mesh = plsc.VectorSubcoreMesh(core_axis_name="c", subcore_axis_name="s")

    @functools.partial(
        pl.kernel,
        compiler_params=_sc_compiler_params(),
        out_type=[
            jax.ShapeDtypeStruct((_NPAD, _D), jnp.float32),
            jax.ShapeDtypeStruct((_NPAD, _D), jnp.float32),
            jax.ShapeDtypeStruct((_NW * _NPAD,), jnp.float32),
        ],
        mesh=mesh,
        scratch_types=[
            pltpu.VMEM((_K,), jnp.int32),
            pltpu.VMEM((_K,), jnp.int32),
            pltpu.VMEM((_K,), jnp.int32),
            pltpu.VMEM((_K,), jnp.int32),
            pltpu.VMEM((_K,), jnp.int32),
            pltpu.VMEM((_K,), jnp.int32),
            pltpu.VMEM((_K,), jnp.int32),
            pltpu.VMEM((_K,), jnp.int32),
            pltpu.VMEM((_K, _D), jnp.float32),
            pltpu.VMEM((_K, _D), jnp.float32),
            pltpu.VMEM((_NPAD,), jnp.float32),
            pltpu.VMEM_SHARED((_NPAD, _D), jnp.float32),
            pltpu.SemaphoreType.DMA,
            pltpu.SemaphoreType.DMA,
            pltpu.SemaphoreType.DMA,
            pltpu.SemaphoreType.DMA,
            pltpu.SemaphoreType.DMA,
            pltpu.SemaphoreType.DMA,
            pltpu.SemaphoreType.DMA,
            pltpu.SemaphoreType.DMA,
            pltpu.SemaphoreType.DMA,
        ],
    )
    def sc_kernel(x_hbm, rowp_hbm, colp_hbm, sum0_out, sum1_out, cnt_out,
                  ridx0, ridx1, ridx2, ridx3, cidx0, cidx1, cidx2, cidx3,
                  gbuf0, gbuf1, hist, acc,
                  si0, si1, si2, si3, sg0, sg1, ss0, ss1, sz):
        c = lax.axis_index("c")
        s = lax.axis_index("s")
        wid = c * _NS + s
        lo = s * _RPT
        cb = wid * _NCHUNK

        zero16 = jnp.zeros((16,), jnp.float32)
        one16 = jnp.ones((16,), jnp.float32)

        ridx = (ridx0, ridx1, ridx2, ridx3)
        cidx = (cidx0, cidx1, cidx2, cidx3)
        si = (si0, si1, si2, si3)
        gset = ((gbuf0, sg0, ss0), (gbuf1, sg1, ss1))

        @pl.loop(0, _K)
        def _(r):
            for q in range(_D // 16):
                gbuf0.at[r, pl.ds(q * 16, 16)][...] = zero16

        for t in range(_RPT // _K):
            pltpu.async_copy(gbuf0, acc.at[pl.ds(lo + t * _K, _K)], sz)

        @pl.loop(0, _NPAD, step=16)
        def _(j):
            hist[pl.ds(j, 16)] = zero16

        for t in range(_RPT // _K):
            pltpu.make_async_copy(gbuf0, acc.at[pl.ds(lo + t * _K, _K)],
                                  sz).wait()

        plsc.subcore_barrier()

        def load_idx(m, sl):
            off = (cb + m) * _K
            pltpu.async_copy(rowp_hbm.at[pl.ds(off, _K)], ridx[sl], si[sl])
            pltpu.async_copy(colp_hbm.at[pl.ds(off, _K)], cidx[sl], si[sl])

        def wait_idx(m, sl):
            off = (cb + m) * _K
            pltpu.make_async_copy(rowp_hbm.at[pl.ds(off, _K)], ridx[sl],
                                  si[sl]).wait()
            pltpu.make_async_copy(colp_hbm.at[pl.ds(off, _K)], cidx[sl],
                                  si[sl]).wait()

        def gather(j, ph):
            gbuf, sg, _ = gset[ph % 2]
            pltpu.async_copy(x_hbm.at[ridx[ph]], gbuf, sg)

        def consume(ph):
            gbuf, sg, ss = gset[ph % 2]
            pltpu.make_async_copy(x_hbm.at[ridx[ph]], gbuf, sg).wait()
            pltpu.async_copy(gbuf, acc.at[cidx[ph]], ss, add=True)
            for q in range(_K // 16):
                idxv = cidx[ph][pl.ds(q * 16, 16)]
                plsc.addupdate_scatter(hist, [idxv], one16)

        def wait_scatter(ph):
            gbuf, _, ss = gset[ph % 2]
            pltpu.make_async_copy(gbuf, acc.at[cidx[ph]], ss).wait()

        pltpu.sync_copy(rowp_hbm.at[pl.ds(cb * _K, _K)], ridx0)
        pltpu.sync_copy(colp_hbm.at[pl.ds(cb * _K, _K)], cidx0)
        pltpu.sync_copy(rowp_hbm.at[pl.ds((cb + 1) * _K, _K)], ridx1)
        pltpu.sync_copy(colp_hbm.at[pl.ds((cb + 1) * _K, _K)], cidx1)
        gather(0, 0)
        load_idx(2, 2)
        gather(1, 1)
        load_idx(3, 3)
        consume(0)
        consume(1)
        wait_scatter(0)
        wait_idx(2, 2)
        gather(2, 2)

        def step(j, ph):
            consume(ph)
            wait_scatter((ph + 3) % 4)
            load_idx(j + 2, (ph + 2) % 4)
            wait_idx(j + 1, (ph + 1) % 4)
            gather(j + 1, (ph + 1) % 4)

        nsteady = (_NCHUNK - 7) // 4

        @pl.loop(0, nsteady)
        def _(t):
            j4 = 4 * t
            for k in range(4):
                step(j4 + 2 + k, (2 + k) % 4)

        step(_NCHUNK - 5, 2)
        step(_NCHUNK - 4, 3)
        step(_NCHUNK - 3, 0)
        consume(1)
        wait_scatter(0)
        wait_idx(_NCHUNK - 1, 2)
        gather(_NCHUNK - 1, 2)
        consume(2)
        wait_scatter(1)
        wait_scatter(2)

        plsc.subcore_barrier()

        pltpu.async_copy(hist, cnt_out.at[pl.ds(wid * _NPAD, _NPAD)], sz)

        def out_slice(j, gbuf):
            pltpu.sync_copy(acc.at[pl.ds(j, _K)], gbuf)

            @pl.when(c == 0)
            def _():
                pltpu.sync_copy(gbuf, sum0_out.at[pl.ds(j, _K)])

            @pl.when(c == 1)
            def _():
                pltpu.sync_copy(gbuf, sum1_out.at[pl.ds(j, _K)])

        @pl.loop(0, _RPT, step=_K)
        def _(j):
            out_slice(lo + j, gbuf0)

        pltpu.make_async_copy(hist, cnt_out.at[pl.ds(wid * _NPAD, _NPAD)],
                              sz).wait()

    return sc_kernel(x, rowp, colp)


def _tc_finish(x_pad, w_t, b2, sum0, sum1, cnt_t):
    blk = 1024

    def body(x_ref, wt_ref, b_ref, s0_ref, s1_ref, c_ref, o_ref):
        ssum = s0_ref[...] + s1_ref[...]
        cc = jnp.sum(c_ref[...], axis=1, keepdims=True)
        m = jnp.where(cc > 0.0, ssum / jnp.maximum(cc, 1.0), x_ref[...])
        o_ref[...] = jnp.dot(
            m, wt_ref[...], preferred_element_type=jnp.float32,
            precision=lax.Precision.HIGHEST,
        ) + b_ref[...]

    return pl.pallas_call(
        body,
        grid=(_NPAD // blk,),
        in_specs=[
            pl.BlockSpec((blk, _D), lambda i: (i, 0)),
            pl.BlockSpec((_D, _D), lambda i: (0, 0)),
            pl.BlockSpec((1, _D), lambda i: (0, 0)),
            pl.BlockSpec((blk, _D), lambda i: (i, 0)),
            pl.BlockSpec((blk, _D), lambda i: (i, 0)),
            pl.BlockSpec((blk, _NW), lambda i: (i, 0)),
        ],
        out_specs=pl.BlockSpec((blk, _D), lambda i: (i, 0)),
        out_shape=jax.ShapeDtypeStruct((_NPAD, _D), jnp.float32),
    )(x_pad, w_t, b2, sum0, sum1, cnt_t)


@jax.jit
def kernel(x, edge_index, W, b):
    row = edge_index[0]
    col = edge_index[1]
    pad = _EPWP - _EPW
    rowp = jnp.pad(row.reshape(_NW, _EPW), ((0, 0), (0, pad))
                   ).reshape(_NW, _NCHUNK, _K)
    colp = jnp.pad(col.reshape(_NW, _EPW), ((0, 0), (0, pad)),
                   constant_values=_NPAD - 1).reshape(_NW, _NCHUNK, _K)
    sum0, sum1, cnth = _sc_segment_sum(x, rowp.reshape(-1), colp.reshape(-1))
    cnt_t = cnth.reshape(_NW, _NPAD).T
    x_pad = jnp.pad(x, ((0, _NPAD - _N), (0, 0)))
    out_pad = _tc_finish(x_pad, W.T, b.reshape(1, _D), sum0, sum1, cnt_t)
    return out_pad[:_N]


# speedup vs baseline: 1.8437x; 1.5950x over previous
import dataclasses
import functools

import jax
import jax.numpy as jnp
from jax import lax
from jax.experimental import pallas as pl
from jax.experimental.pallas import tpu as pltpu
from jax.experimental.pallas import tpu_sc as plsc

_N = 10000
_E = 320000
_D = 128
_NC = 2
_NS = 16
_NW = _NC * _NS
_NPAD = 10240
_EPW = _E // _NW
_K = 80
_NCHUNK = -(-_EPW // _K)
_EPWP = _NCHUNK * _K
_RPT = _NPAD // _NS


def _sc_compiler_params():
    cp = pltpu.CompilerParams()
    if "needs_layout_passes" in pltpu.CompilerParams.__dataclass_fields__:
        cp = dataclasses.replace(cp, needs_layout_passes=False)
    return cp


def _sc_segment_sum(x, rowp, colp):
    mesh = plsc.VectorSubcoreMesh(core_axis_name="c", subcore_axis_name="s")

    @functools.partial(
        pl.kernel,
        compiler_params=_sc_compiler_params(),
        out_type=[
            jax.ShapeDtypeStruct((_NPAD, _D), jnp.float32),
            jax.ShapeDtypeStruct((_NPAD, _D), jnp.float32),
            jax.ShapeDtypeStruct((_NW * _NPAD,), jnp.float32),
        ],
        mesh=mesh,
        scratch_types=[
            pltpu.VMEM((_K,), jnp.int32),
            pltpu.VMEM((_K,), jnp.int32),
            pltpu.VMEM((_K,), jnp.int32),
            pltpu.VMEM((_K,), jnp.int32),
            pltpu.VMEM((_K, _D), jnp.float32),
            pltpu.VMEM((_K, _D), jnp.float32),
            pltpu.VMEM((_NPAD,), jnp.float32),
            pltpu.VMEM_SHARED((_NPAD, _D), jnp.float32),
            pltpu.SemaphoreType.DMA,
            pltpu.SemaphoreType.DMA,
            pltpu.SemaphoreType.DMA,
        ],
    )
    def sc_kernel(x_hbm, rowp_hbm, colp_hbm, sum0_out, sum1_out, cnt_out,
                  ridx0, cidx0, ridx1, cidx1, gbuf0, gbuf1, hist, acc,
                  sg0, sg1, sz):
        c = lax.axis_index("c")
        s = lax.axis_index("s")
        wid = c * _NS + s
        lo = s * _RPT
        cb = wid * _NCHUNK

        zero16 = jnp.zeros((16,), jnp.float32)
        one16 = jnp.ones((16,), jnp.float32)

        @pl.loop(0, _K)
        def _(r):
            for q in range(_D // 16):
                gbuf0.at[r, pl.ds(q * 16, 16)][...] = zero16

        for t in range(_RPT // _K):
            pltpu.async_copy(gbuf0, acc.at[pl.ds(lo + t * _K, _K)], sz)

        @pl.loop(0, _NPAD, step=16)
        def _(j):
            hist[pl.ds(j, 16)] = zero16

        for t in range(_RPT // _K):
            pltpu.make_async_copy(gbuf0, acc.at[pl.ds(lo + t * _K, _K)],
                                  sz).wait()

        plsc.subcore_barrier()

        def load_idx(j, ridx, cidx):
            off = (cb + j) * _K
            pltpu.async_copy(rowp_hbm.at[pl.ds(off, _K)], ridx, sz)
            pltpu.async_copy(colp_hbm.at[pl.ds(off, _K)], cidx, sz)
            pltpu.make_async_copy(rowp_hbm.at[pl.ds(off, _K)], ridx,
                                  sz).wait()
            pltpu.make_async_copy(colp_hbm.at[pl.ds(off, _K)], cidx,
                                  sz).wait()

        def consume(ridx, cidx, gbuf, sg):
            pltpu.make_async_copy(x_hbm.at[ridx], gbuf, sg).wait()
            pltpu.sync_copy(gbuf, acc.at[cidx], add=True)
            for q in range(_K // 16):
                idxv = cidx[pl.ds(q * 16, 16)]
                plsc.addupdate_scatter(hist, [idxv], one16)

        load_idx(0, ridx0, cidx0)
        pltpu.async_copy(x_hbm.at[ridx0], gbuf0, sg0)
        load_idx(1, ridx1, cidx1)
        pltpu.async_copy(x_hbm.at[ridx1], gbuf1, sg1)

        @pl.loop(0, (_NCHUNK - 3) // 2)
        def _(h):
            consume(ridx0, cidx0, gbuf0, sg0)
            load_idx(2 * h + 2, ridx0, cidx0)
            pltpu.async_copy(x_hbm.at[ridx0], gbuf0, sg0)
            consume(ridx1, cidx1, gbuf1, sg1)
            load_idx(2 * h + 3, ridx1, cidx1)
            pltpu.async_copy(x_hbm.at[ridx1], gbuf1, sg1)

        consume(ridx0, cidx0, gbuf0, sg0)
        load_idx(_NCHUNK - 1, ridx0, cidx0)
        pltpu.async_copy(x_hbm.at[ridx0], gbuf0, sg0)
        consume(ridx1, cidx1, gbuf1, sg1)
        consume(ridx0, cidx0, gbuf0, sg0)

        plsc.subcore_barrier()

        pltpu.async_copy(hist, cnt_out.at[pl.ds(wid * _NPAD, _NPAD)], sz)

        def out_slice(j, gbuf):
            pltpu.sync_copy(acc.at[pl.ds(j, _K)], gbuf)

            @pl.when(c == 0)
            def _():
                pltpu.sync_copy(gbuf, sum0_out.at[pl.ds(j, _K)])

            @pl.when(c == 1)
            def _():
                pltpu.sync_copy(gbuf, sum1_out.at[pl.ds(j, _K)])

        @pl.loop(0, _RPT, step=_K)
        def _(j):
            out_slice(lo + j, gbuf0)

        pltpu.make_async_copy(hist, cnt_out.at[pl.ds(wid * _NPAD, _NPAD)],
                              sz).wait()

    return sc_kernel(x, rowp, colp)


def _tc_finish(x_pad, w_t, b2, sum0, sum1, cnt_t):
    blk = 1024

    def body(x_ref, wt_ref, b_ref, s0_ref, s1_ref, c_ref, o_ref):
        ssum = s0_ref[...] + s1_ref[...]
        cc = jnp.sum(c_ref[...], axis=1, keepdims=True)
        m = jnp.where(cc > 0.0, ssum / jnp.maximum(cc, 1.0), x_ref[...])
        o_ref[...] = jnp.dot(
            m, wt_ref[...], preferred_element_type=jnp.float32,
            precision=lax.Precision.HIGHEST,
        ) + b_ref[...]

    return pl.pallas_call(
        body,
        grid=(_NPAD // blk,),
        in_specs=[
            pl.BlockSpec((blk, _D), lambda i: (i, 0)),
            pl.BlockSpec((_D, _D), lambda i: (0, 0)),
            pl.BlockSpec((1, _D), lambda i: (0, 0)),
            pl.BlockSpec((blk, _D), lambda i: (i, 0)),
            pl.BlockSpec((blk, _D), lambda i: (i, 0)),
            pl.BlockSpec((blk, _NW), lambda i: (i, 0)),
        ],
        out_specs=pl.BlockSpec((blk, _D), lambda i: (i, 0)),
        out_shape=jax.ShapeDtypeStruct((_NPAD, _D), jnp.float32),
    )(x_pad, w_t, b2, sum0, sum1, cnt_t)


@jax.jit
def kernel(x, edge_index, W, b):
    row = edge_index[0]
    col = edge_index[1]
    pad = _EPWP - _EPW
    rowp = jnp.pad(row.reshape(_NW, _EPW), ((0, 0), (0, pad))
                   ).reshape(_NW, _NCHUNK, _K)
    colp = jnp.pad(col.reshape(_NW, _EPW), ((0, 0), (0, pad)),
                   constant_values=_NPAD - 1).reshape(_NW, _NCHUNK, _K)
    sum0, sum1, cnth = _sc_segment_sum(x, rowp.reshape(-1), colp.reshape(-1))
    cnt_t = cnth.reshape(_NW, _NPAD).T
    x_pad = jnp.pad(x, ((0, _NPAD - _N), (0, 0)))
    out_pad = _tc_finish(x_pad, W.T, b.reshape(1, _D), sum0, sum1, cnt_t)
    return out_pad[:_N]
